# baseline (speedup 1.0000x reference)
import math

import jax
import jax.numpy as jnp
from jax import lax
from jax.experimental import pallas as pl
from jax.experimental.pallas import tpu as pltpu
from jax.experimental.pallas import tpu_sc as plsc

VOCAB = 1000000
D_MODEL = 64
SCALE = math.sqrt(D_MODEL)

_INFO = plsc.get_sparse_core_info()
NC, NS, L = _INFO.num_cores, _INFO.num_subcores, _INFO.num_lanes
NW = NC * NS

B_TOTAL = 4096 * 50
B_PER_W = B_TOTAL // NW
CHUNK = 128
N_CHUNKS = B_PER_W // CHUNK
NBUF = 5
N_BLOCKS = N_CHUNKS // NBUF


def _sc_body(idx_hbm, table_hbm, out_hbm, idx_v, *rest):
    gbufs = rest[0:NBUF]
    obufs = rest[NBUF:2 * NBUF]
    gsems = rest[2 * NBUF:3 * NBUF]
    osems = rest[3 * NBUF:4 * NBUF]

    wid = lax.axis_index("s") * NC + lax.axis_index("c")
    base = wid * B_PER_W

    pltpu.sync_copy(idx_hbm.at[wid], idx_v)

    def start_gather(j, b):
        pltpu.async_copy(table_hbm.at[idx_v.at[j]], gbufs[b], gsems[b])

    def wait_gather(j, b):
        pltpu.make_async_copy(table_hbm.at[idx_v.at[j]], gbufs[b],
                              gsems[b]).wait()

    def out_slice(j):
        return out_hbm.at[pl.ds(base + j * CHUNK, CHUNK)]

    def start_out(j, b):
        pltpu.async_copy(obufs[b], out_slice(j), osems[b])

    def wait_out(j, b):
        pltpu.make_async_copy(obufs[b], out_slice(j), osems[b]).wait()

    def scale(b):
        gb, ob = gbufs[b], obufs[b]

        def row_step(r, _):
            for c in range(D_MODEL // L):
                sl = pl.ds(c * L, L)
                ob[r, sl] = gb[r, sl] * SCALE
            return 0

        lax.fori_loop(0, CHUNK, row_step, 0, unroll=4)

    for b in range(NBUF):
        start_gather(b, b)

    for b in range(NBUF):
        wait_gather(b, b)
        scale(b)
        start_out(b, b)
        start_gather(b + NBUF, b)

    def block(i, _):
        j0 = i * NBUF
        for b in range(NBUF):
            j = j0 + b
            wait_gather(j, b)
            wait_out(j - NBUF, b)
            scale(b)
            start_out(j, b)
            start_gather(j + NBUF, b)
        return 0

    lax.fori_loop(1, N_BLOCKS - 1, block, 0)

    j0 = (N_BLOCKS - 1) * NBUF
    for b in range(NBUF):
        j = j0 + b
        wait_gather(j, b)
        wait_out(j - NBUF, b)
        scale(b)
        start_out(j, b)

    for b in range(NBUF):
        wait_out(j0 + b, b)


def kernel(x, table):
    idx3d = x.reshape(NW, N_CHUNKS, CHUNK).astype(jnp.int32)
    mesh = plsc.VectorSubcoreMesh(core_axis_name="c", subcore_axis_name="s")
    scratch = [pltpu.VMEM((N_CHUNKS, CHUNK), jnp.int32)]
    scratch += [pltpu.VMEM((CHUNK, D_MODEL), jnp.float32)
                for _ in range(2 * NBUF)]
    scratch += [pltpu.SemaphoreType.DMA for _ in range(2 * NBUF)]
    sc_call = pl.kernel(
        _sc_body,
        mesh=mesh,
        out_type=jax.ShapeDtypeStruct((B_TOTAL, D_MODEL), jnp.float32),
        scratch_types=scratch,
        compiler_params=pltpu.CompilerParams(use_tc_tiling_on_sc=False),
    )
    out = sc_call(idx3d, table)
    return out.reshape(x.shape[0], x.shape[1], D_MODEL)
